# baseline (speedup 1.0000x reference)
import jax
import jax.numpy as jnp
from jax.experimental import pallas as pl
from jax.experimental.pallas import tpu as pltpu

B, L, D = 8, 256, 384
E, N, R = 768, 16, 24
HW = 16
EPS = 1e-5
FILL_CHUNK = 16
UNROLL = 32
GROUPS = L // UNROLL


def _vssm_kernel(x_ref, w_in_x_ref, w_in_z_ref, wx_dt_ref, wx_b_ref,
                 wx_c_ref, w_dt_t_ref, b_dt_ref, a_log_t_ref, d_param_ref,
                 w_out_t_ref, gamma_ref, beta_ref, o_ref,
                 xin_s, z_s, dt_s, bp_s, cp_s, cpx_s, a_s,
                 abar_s, bbar_s, y0_s, y1_s, y2_s, y3_s):
    a_s[...] = -jnp.exp(a_log_t_ref[...])

    def batch_body(b, _):
        xb = x_ref[b]

        xin = jnp.dot(xb, w_in_x_ref[...], preferred_element_type=jnp.float32)
        xin_s[...] = xin
        z_s[...] = jnp.dot(xb, w_in_z_ref[...],
                           preferred_element_type=jnp.float32)
        dt_in = jnp.dot(xin, wx_dt_ref[...],
                        preferred_element_type=jnp.float32)
        bp_s[...] = jnp.dot(xin, wx_b_ref[...],
                            preferred_element_type=jnp.float32)
        cp_s[...] = jnp.dot(xin, wx_c_ref[...],
                            preferred_element_type=jnp.float32)
        cpx_s[...] = cp_s[...].reshape(L, 1, N)
        dt_s[...] = jnp.dot(dt_in, w_dt_t_ref[...],
                            preferred_element_type=jnp.float32) + b_dt_ref[...]

        def fill_body(i, _):
            sl = pl.ds(i * FILL_CHUNK, FILL_CHUNK)
            dt_r = dt_s[sl, :]
            dt_c = (jnp.maximum(dt_r, 0.0) +
                    jnp.log1p(jnp.exp(-jnp.abs(dt_r))))
            xin_c = xin_s[sl, :]
            bp_c = bp_s[sl, :]
            a_v = a_s[...]
            abar_s[sl] = jnp.exp(dt_c[:, None, :] * a_v[None, :, :])
            bbar_s[sl] = (dt_c * xin_c)[:, None, :] * bp_c[:, :, None]
            return 0

        jax.lax.fori_loop(0, L // FILL_CHUNK, fill_body, 0)

        perm = lambda t: ((t & (HW - 1)) << 4) | (t >> 4)

        def run_dir_pair(ya_ref, load_a, yb_ref, load_b):
            def body(i, hs):
                ha, hb = hs
                fwd_base = pl.multiple_of(UNROLL * i, UNROLL)
                rev_base = pl.multiple_of((L - UNROLL) - UNROLL * i, UNROLL)
                for k in range(UNROLL):
                    la = load_a(i, k)
                    lb = load_b(i, k)
                    ha = abar_s[la] * ha + bbar_s[la]
                    hb = abar_s[lb] * hb + bbar_s[lb]
                    red_a = jnp.dot(cpx_s[la], ha,
                                    preferred_element_type=jnp.float32)
                    red_b = jnp.dot(cpx_s[lb], hb,
                                    preferred_element_type=jnp.float32)
                    ya_ref[pl.ds(fwd_base + k, 1), :] = red_a
                    yb_ref[pl.ds(rev_base + (UNROLL - 1 - k), 1), :] = red_b
                return (ha, hb)

            z0 = jnp.zeros((N, E), jnp.float32)
            jax.lax.fori_loop(0, GROUPS, body, (z0, z0))

        run_dir_pair(y0_s, lambda i, k: UNROLL * i + k,
                     y1_s, lambda i, k: (L - 1) - (UNROLL * i + k))
        run_dir_pair(y2_s, lambda i, k: perm(UNROLL * i + k),
                     y3_s, lambda i, k: perm((L - 1) - (UNROLL * i + k)))

        t2 = jnp.swapaxes(y2_s[...].reshape(HW, HW, E), 0, 1).reshape(L, E)
        t3 = jnp.swapaxes(y3_s[...].reshape(HW, HW, E), 0, 1).reshape(L, E)
        ysum = y0_s[...] + y1_s[...] + t2 + t3

        z = z_s[...]
        sig = 1.0 / (1.0 + jnp.exp(-z))
        y = ysum * 0.25 * (z * sig) + xin_s[...] * d_param_ref[...]
        out = jnp.dot(y, w_out_t_ref[...],
                      preferred_element_type=jnp.float32) + xb
        mu = jnp.mean(out, axis=-1, keepdims=True)
        xc = out - mu
        var = jnp.mean(xc * xc, axis=-1, keepdims=True)
        o_ref[b] = (xc * jax.lax.rsqrt(var + EPS) * gamma_ref[...] +
                    beta_ref[...])
        return 0

    jax.lax.fori_loop(0, B, batch_body, 0)


@jax.jit
def kernel(x, W_in, A_log, W_x, W_dt, b_dt, D_param, W_out, gamma, beta):
    w_in_x = W_in[:E].T
    w_in_z = W_in[E:].T
    wx_dt = W_x[:R].T
    wx_b = W_x[R:R + N].T
    wx_c = W_x[R + N:].T
    w_dt_t = W_dt.T
    a_log_t = A_log.T
    w_out_t = W_out.T

    vmem = pl.BlockSpec(memory_space=pltpu.VMEM)
    f32 = jnp.float32
    return pl.pallas_call(
        _vssm_kernel,
        out_shape=jax.ShapeDtypeStruct((B, L, D), f32),
        in_specs=[vmem] * 13,
        out_specs=vmem,
        scratch_shapes=[
            pltpu.VMEM((L, E), f32),
            pltpu.VMEM((L, E), f32),
            pltpu.VMEM((L, E), f32),
            pltpu.VMEM((L, N), f32),
            pltpu.VMEM((L, N), f32),
            pltpu.VMEM((L, 1, N), f32),
            pltpu.VMEM((N, E), f32),
            pltpu.VMEM((L, N, E), f32),
            pltpu.VMEM((L, N, E), f32),
            pltpu.VMEM((L, E), f32),
            pltpu.VMEM((L, E), f32),
            pltpu.VMEM((L, E), f32),
            pltpu.VMEM((L, E), f32),
        ],
        compiler_params=pltpu.CompilerParams(
            vmem_limit_bytes=56 * 1024 * 1024,
        ),
        name="vssm_quad_scan",
    )(x, w_in_x, w_in_z, wx_dt, wx_b, wx_c, w_dt_t, b_dt.reshape(1, E),
      a_log_t, D_param.reshape(1, E), w_out_t, gamma.reshape(1, D),
      beta.reshape(1, D))


# speedup vs baseline: 1.0057x; 1.0057x over previous
import jax
import jax.numpy as jnp
from jax.experimental import pallas as pl
from jax.experimental.pallas import tpu as pltpu

B, L, D = 8, 256, 384
E, N, R = 768, 16, 24
HW = 16
EPS = 1e-5
FILL_CHUNK = 16
UNROLL = 32
GROUPS = L // UNROLL


def _vssm_kernel(x_ref, w_in_x_ref, w_in_z_ref, wx_dt_ref, wx_b_ref,
                 wx_c_ref, w_dt_t_ref, b_dt_ref, a_log_t_ref, d_param_ref,
                 w_out_t_ref, gamma_ref, beta_ref, o_ref,
                 xin_s, z_s, dt_s, bp_s, cp_s, cpx_s, a_s,
                 abar_s, bbar_s, y0_s, y1_s, y2_s, y3_s):
    a_s[...] = -jnp.exp(a_log_t_ref[...])

    def batch_body(b, _):
        xb = x_ref[b]

        xin = jnp.dot(xb, w_in_x_ref[...], preferred_element_type=jnp.float32)
        xin_s[...] = xin
        z_s[...] = jnp.dot(xb, w_in_z_ref[...],
                           preferred_element_type=jnp.float32)
        dt_in = jnp.dot(xin, wx_dt_ref[...],
                        preferred_element_type=jnp.float32)
        bp_s[...] = jnp.dot(xin, wx_b_ref[...],
                            preferred_element_type=jnp.float32)
        cp_s[...] = jnp.dot(xin, wx_c_ref[...],
                            preferred_element_type=jnp.float32)
        cpx_s[...] = cp_s[...].reshape(L, 1, N)
        dt_s[...] = jnp.dot(dt_in, w_dt_t_ref[...],
                            preferred_element_type=jnp.float32) + b_dt_ref[...]

        def fill_body(i, _):
            sl = pl.ds(i * FILL_CHUNK, FILL_CHUNK)
            dt_r = dt_s[sl, :]
            dt_c = (jnp.maximum(dt_r, 0.0) +
                    jnp.log1p(jnp.exp(-jnp.abs(dt_r))))
            xin_c = xin_s[sl, :]
            bp_c = bp_s[sl, :]
            a_v = a_s[...]
            abar_s[sl] = jnp.exp(dt_c[:, None, :] * a_v[None, :, :])
            bbar_s[sl] = (dt_c * xin_c)[:, None, :] * bp_c[:, :, None]
            return 0

        jax.lax.fori_loop(0, L // FILL_CHUNK, fill_body, 0)

        perm = lambda t: ((t & (HW - 1)) << 4) | (t >> 4)
        loaders = (lambda i, k: UNROLL * i + k,
                   lambda i, k: (L - 1) - (UNROLL * i + k),
                   lambda i, k: perm(UNROLL * i + k),
                   lambda i, k: perm((L - 1) - (UNROLL * i + k)))
        y_refs = (y0_s, y1_s, y2_s, y3_s)

        def body(i, hs):
            hs = list(hs)
            fwd_base = pl.multiple_of(UNROLL * i, UNROLL)
            rev_base = pl.multiple_of((L - UNROLL) - UNROLL * i, UNROLL)
            for k in range(UNROLL):
                for d in range(4):
                    l = loaders[d](i, k)
                    hs[d] = abar_s[l] * hs[d] + bbar_s[l]
                    red = jnp.dot(cpx_s[l], hs[d],
                                  preferred_element_type=jnp.float32)
                    if d % 2 == 0:
                        y_refs[d][pl.ds(fwd_base + k, 1), :] = red
                    else:
                        y_refs[d][pl.ds(rev_base + (UNROLL - 1 - k), 1),
                                  :] = red
            return tuple(hs)

        z0 = jnp.zeros((N, E), jnp.float32)
        jax.lax.fori_loop(0, GROUPS, body, (z0, z0, z0, z0))

        t2 = jnp.swapaxes(y2_s[...].reshape(HW, HW, E), 0, 1).reshape(L, E)
        t3 = jnp.swapaxes(y3_s[...].reshape(HW, HW, E), 0, 1).reshape(L, E)
        ysum = y0_s[...] + y1_s[...] + t2 + t3

        z = z_s[...]
        sig = 1.0 / (1.0 + jnp.exp(-z))
        y = ysum * 0.25 * (z * sig) + xin_s[...] * d_param_ref[...]
        out = jnp.dot(y, w_out_t_ref[...],
                      preferred_element_type=jnp.float32) + xb
        mu = jnp.mean(out, axis=-1, keepdims=True)
        xc = out - mu
        var = jnp.mean(xc * xc, axis=-1, keepdims=True)
        o_ref[b] = (xc * jax.lax.rsqrt(var + EPS) * gamma_ref[...] +
                    beta_ref[...])
        return 0

    jax.lax.fori_loop(0, B, batch_body, 0)


@jax.jit
def kernel(x, W_in, A_log, W_x, W_dt, b_dt, D_param, W_out, gamma, beta):
    w_in_x = W_in[:E].T
    w_in_z = W_in[E:].T
    wx_dt = W_x[:R].T
    wx_b = W_x[R:R + N].T
    wx_c = W_x[R + N:].T
    w_dt_t = W_dt.T
    a_log_t = A_log.T
    w_out_t = W_out.T

    vmem = pl.BlockSpec(memory_space=pltpu.VMEM)
    f32 = jnp.float32
    return pl.pallas_call(
        _vssm_kernel,
        out_shape=jax.ShapeDtypeStruct((B, L, D), f32),
        in_specs=[vmem] * 13,
        out_specs=vmem,
        scratch_shapes=[
            pltpu.VMEM((L, E), f32),
            pltpu.VMEM((L, E), f32),
            pltpu.VMEM((L, E), f32),
            pltpu.VMEM((L, N), f32),
            pltpu.VMEM((L, N), f32),
            pltpu.VMEM((L, 1, N), f32),
            pltpu.VMEM((N, E), f32),
            pltpu.VMEM((L, N, E), f32),
            pltpu.VMEM((L, N, E), f32),
            pltpu.VMEM((L, E), f32),
            pltpu.VMEM((L, E), f32),
            pltpu.VMEM((L, E), f32),
            pltpu.VMEM((L, E), f32),
        ],
        compiler_params=pltpu.CompilerParams(
            vmem_limit_bytes=56 * 1024 * 1024,
        ),
        name="vssm_quad_scan",
    )(x, w_in_x, w_in_z, wx_dt, wx_b, wx_c, w_dt_t, b_dt.reshape(1, E),
      a_log_t, D_param.reshape(1, E), w_out_t, gamma.reshape(1, D),
      beta.reshape(1, D))


# speedup vs baseline: 1.0261x; 1.0203x over previous
import jax
import jax.numpy as jnp
from jax.experimental import pallas as pl
from jax.experimental.pallas import tpu as pltpu

B, L, D = 8, 256, 384
E, N, R = 768, 16, 24
HW = 16
EPS = 1e-5
FILL_CHUNK = 16
UNROLL = 64
GROUPS = L // UNROLL


def _vssm_kernel(x_ref, w_in_x_ref, w_in_z_ref, wx_dt_ref, wx_b_ref,
                 wx_c_ref, w_dt_t_ref, b_dt_ref, a_log_t_ref, d_param_ref,
                 w_out_t_ref, gamma_ref, beta_ref, o_ref,
                 xin_s, z_s, dt_s, bp_s, cp_s, cpx_s, a_s,
                 abar_s, bbar_s, y0_s, y1_s, y2_s, y3_s):
    a_s[...] = -jnp.exp(a_log_t_ref[...])

    def batch_body(b, _):
        xb = x_ref[b]

        xin = jnp.dot(xb, w_in_x_ref[...], preferred_element_type=jnp.float32)
        xin_s[...] = xin
        z_s[...] = jnp.dot(xb, w_in_z_ref[...],
                           preferred_element_type=jnp.float32)
        dt_in = jnp.dot(xin, wx_dt_ref[...],
                        preferred_element_type=jnp.float32)
        bp_s[...] = jnp.dot(xin, wx_b_ref[...],
                            preferred_element_type=jnp.float32)
        cp_s[...] = jnp.dot(xin, wx_c_ref[...],
                            preferred_element_type=jnp.float32)
        cpx_s[...] = cp_s[...].reshape(L, 1, N)
        dt_s[...] = jnp.dot(dt_in, w_dt_t_ref[...],
                            preferred_element_type=jnp.float32) + b_dt_ref[...]

        def fill_body(i, _):
            sl = pl.ds(i * FILL_CHUNK, FILL_CHUNK)
            dt_r = dt_s[sl, :]
            dt_c = (jnp.maximum(dt_r, 0.0) +
                    jnp.log1p(jnp.exp(-jnp.abs(dt_r))))
            xin_c = xin_s[sl, :]
            bp_c = bp_s[sl, :]
            a_v = a_s[...]
            abar_s[sl] = jnp.exp(dt_c[:, None, :] * a_v[None, :, :])
            bbar_s[sl] = (dt_c * xin_c)[:, None, :] * bp_c[:, :, None]
            return 0

        jax.lax.fori_loop(0, L // FILL_CHUNK, fill_body, 0)

        perm = lambda t: ((t & (HW - 1)) << 4) | (t >> 4)
        loaders = (lambda i, k: UNROLL * i + k,
                   lambda i, k: (L - 1) - (UNROLL * i + k),
                   lambda i, k: perm(UNROLL * i + k),
                   lambda i, k: perm((L - 1) - (UNROLL * i + k)))
        y_refs = (y0_s, y1_s, y2_s, y3_s)

        def body(i, hs):
            hs = list(hs)
            fwd_base = pl.multiple_of(UNROLL * i, UNROLL)
            rev_base = pl.multiple_of((L - UNROLL) - UNROLL * i, UNROLL)
            for k in range(UNROLL):
                for d in range(4):
                    l = loaders[d](i, k)
                    hs[d] = abar_s[l] * hs[d] + bbar_s[l]
                    red = jnp.dot(cpx_s[l], hs[d],
                                  preferred_element_type=jnp.float32)
                    if d % 2 == 0:
                        y_refs[d][pl.ds(fwd_base + k, 1), :] = red
                    else:
                        y_refs[d][pl.ds(rev_base + (UNROLL - 1 - k), 1),
                                  :] = red
            return tuple(hs)

        z0 = jnp.zeros((N, E), jnp.float32)
        jax.lax.fori_loop(0, GROUPS, body, (z0, z0, z0, z0))

        t2 = jnp.swapaxes(y2_s[...].reshape(HW, HW, E), 0, 1).reshape(L, E)
        t3 = jnp.swapaxes(y3_s[...].reshape(HW, HW, E), 0, 1).reshape(L, E)
        ysum = y0_s[...] + y1_s[...] + t2 + t3

        z = z_s[...]
        sig = 1.0 / (1.0 + jnp.exp(-z))
        y = ysum * 0.25 * (z * sig) + xin_s[...] * d_param_ref[...]
        out = jnp.dot(y, w_out_t_ref[...],
                      preferred_element_type=jnp.float32) + xb
        mu = jnp.mean(out, axis=-1, keepdims=True)
        xc = out - mu
        var = jnp.mean(xc * xc, axis=-1, keepdims=True)
        o_ref[b] = (xc * jax.lax.rsqrt(var + EPS) * gamma_ref[...] +
                    beta_ref[...])
        return 0

    jax.lax.fori_loop(0, B, batch_body, 0)


@jax.jit
def kernel(x, W_in, A_log, W_x, W_dt, b_dt, D_param, W_out, gamma, beta):
    w_in_x = W_in[:E].T
    w_in_z = W_in[E:].T
    wx_dt = W_x[:R].T
    wx_b = W_x[R:R + N].T
    wx_c = W_x[R + N:].T
    w_dt_t = W_dt.T
    a_log_t = A_log.T
    w_out_t = W_out.T

    vmem = pl.BlockSpec(memory_space=pltpu.VMEM)
    f32 = jnp.float32
    return pl.pallas_call(
        _vssm_kernel,
        out_shape=jax.ShapeDtypeStruct((B, L, D), f32),
        in_specs=[vmem] * 13,
        out_specs=vmem,
        scratch_shapes=[
            pltpu.VMEM((L, E), f32),
            pltpu.VMEM((L, E), f32),
            pltpu.VMEM((L, E), f32),
            pltpu.VMEM((L, N), f32),
            pltpu.VMEM((L, N), f32),
            pltpu.VMEM((L, 1, N), f32),
            pltpu.VMEM((N, E), f32),
            pltpu.VMEM((L, N, E), f32),
            pltpu.VMEM((L, N, E), f32),
            pltpu.VMEM((L, E), f32),
            pltpu.VMEM((L, E), f32),
            pltpu.VMEM((L, E), f32),
            pltpu.VMEM((L, E), f32),
        ],
        compiler_params=pltpu.CompilerParams(
            vmem_limit_bytes=56 * 1024 * 1024,
        ),
        name="vssm_quad_scan",
    )(x, w_in_x, w_in_z, wx_dt, wx_b, wx_c, w_dt_t, b_dt.reshape(1, E),
      a_log_t, D_param.reshape(1, E), w_out_t, gamma.reshape(1, D),
      beta.reshape(1, D))


# speedup vs baseline: 1.0296x; 1.0033x over previous
import jax
import jax.numpy as jnp
from jax.experimental import pallas as pl
from jax.experimental.pallas import tpu as pltpu

B, L, D = 8, 256, 384
E, N, R = 768, 16, 24
HW = 16
EPS = 1e-5
FILL_CHUNK = 16
UNROLL = 128
GROUPS = L // UNROLL


def _vssm_kernel(x_ref, w_in_x_ref, w_in_z_ref, wx_dt_ref, wx_b_ref,
                 wx_c_ref, w_dt_t_ref, b_dt_ref, a_log_t_ref, d_param_ref,
                 w_out_t_ref, gamma_ref, beta_ref, o_ref,
                 xin_s, z_s, dt_s, bp_s, cp_s, cpx_s, a_s,
                 abar_s, bbar_s, y0_s, y1_s, y2_s, y3_s):
    a_s[...] = -jnp.exp(a_log_t_ref[...])

    def batch_body(b, _):
        xb = x_ref[b]

        xin = jnp.dot(xb, w_in_x_ref[...], preferred_element_type=jnp.float32)
        xin_s[...] = xin
        z_s[...] = jnp.dot(xb, w_in_z_ref[...],
                           preferred_element_type=jnp.float32)
        dt_in = jnp.dot(xin, wx_dt_ref[...],
                        preferred_element_type=jnp.float32)
        bp_s[...] = jnp.dot(xin, wx_b_ref[...],
                            preferred_element_type=jnp.float32)
        cp_s[...] = jnp.dot(xin, wx_c_ref[...],
                            preferred_element_type=jnp.float32)
        cpx_s[...] = cp_s[...].reshape(L, 1, N)
        dt_s[...] = jnp.dot(dt_in, w_dt_t_ref[...],
                            preferred_element_type=jnp.float32) + b_dt_ref[...]

        def fill_body(i, _):
            sl = pl.ds(i * FILL_CHUNK, FILL_CHUNK)
            dt_r = dt_s[sl, :]
            dt_c = (jnp.maximum(dt_r, 0.0) +
                    jnp.log1p(jnp.exp(-jnp.abs(dt_r))))
            xin_c = xin_s[sl, :]
            bp_c = bp_s[sl, :]
            a_v = a_s[...]
            abar_s[sl] = jnp.exp(dt_c[:, None, :] * a_v[None, :, :])
            bbar_s[sl] = (dt_c * xin_c)[:, None, :] * bp_c[:, :, None]
            return 0

        jax.lax.fori_loop(0, L // FILL_CHUNK, fill_body, 0)

        perm = lambda t: ((t & (HW - 1)) << 4) | (t >> 4)
        loaders = (lambda i, k: UNROLL * i + k,
                   lambda i, k: (L - 1) - (UNROLL * i + k),
                   lambda i, k: perm(UNROLL * i + k),
                   lambda i, k: perm((L - 1) - (UNROLL * i + k)))
        y_refs = (y0_s, y1_s, y2_s, y3_s)

        def body(i, hs):
            hs = list(hs)
            fwd_base = pl.multiple_of(UNROLL * i, UNROLL)
            rev_base = pl.multiple_of((L - UNROLL) - UNROLL * i, UNROLL)
            for k in range(UNROLL):
                for d in range(4):
                    l = loaders[d](i, k)
                    hs[d] = abar_s[l] * hs[d] + bbar_s[l]
                    red = jnp.dot(cpx_s[l], hs[d],
                                  preferred_element_type=jnp.float32)
                    if d % 2 == 0:
                        y_refs[d][pl.ds(fwd_base + k, 1), :] = red
                    else:
                        y_refs[d][pl.ds(rev_base + (UNROLL - 1 - k), 1),
                                  :] = red
            return tuple(hs)

        z0 = jnp.zeros((N, E), jnp.float32)
        jax.lax.fori_loop(0, GROUPS, body, (z0, z0, z0, z0))

        t2 = jnp.swapaxes(y2_s[...].reshape(HW, HW, E), 0, 1).reshape(L, E)
        t3 = jnp.swapaxes(y3_s[...].reshape(HW, HW, E), 0, 1).reshape(L, E)
        ysum = y0_s[...] + y1_s[...] + t2 + t3

        z = z_s[...]
        sig = 1.0 / (1.0 + jnp.exp(-z))
        y = ysum * 0.25 * (z * sig) + xin_s[...] * d_param_ref[...]
        out = jnp.dot(y, w_out_t_ref[...],
                      preferred_element_type=jnp.float32) + xb
        mu = jnp.mean(out, axis=-1, keepdims=True)
        xc = out - mu
        var = jnp.mean(xc * xc, axis=-1, keepdims=True)
        o_ref[b] = (xc * jax.lax.rsqrt(var + EPS) * gamma_ref[...] +
                    beta_ref[...])
        return 0

    jax.lax.fori_loop(0, B, batch_body, 0)


@jax.jit
def kernel(x, W_in, A_log, W_x, W_dt, b_dt, D_param, W_out, gamma, beta):
    w_in_x = W_in[:E].T
    w_in_z = W_in[E:].T
    wx_dt = W_x[:R].T
    wx_b = W_x[R:R + N].T
    wx_c = W_x[R + N:].T
    w_dt_t = W_dt.T
    a_log_t = A_log.T
    w_out_t = W_out.T

    vmem = pl.BlockSpec(memory_space=pltpu.VMEM)
    f32 = jnp.float32
    return pl.pallas_call(
        _vssm_kernel,
        out_shape=jax.ShapeDtypeStruct((B, L, D), f32),
        in_specs=[vmem] * 13,
        out_specs=vmem,
        scratch_shapes=[
            pltpu.VMEM((L, E), f32),
            pltpu.VMEM((L, E), f32),
            pltpu.VMEM((L, E), f32),
            pltpu.VMEM((L, N), f32),
            pltpu.VMEM((L, N), f32),
            pltpu.VMEM((L, 1, N), f32),
            pltpu.VMEM((N, E), f32),
            pltpu.VMEM((L, N, E), f32),
            pltpu.VMEM((L, N, E), f32),
            pltpu.VMEM((L, E), f32),
            pltpu.VMEM((L, E), f32),
            pltpu.VMEM((L, E), f32),
            pltpu.VMEM((L, E), f32),
        ],
        compiler_params=pltpu.CompilerParams(
            vmem_limit_bytes=56 * 1024 * 1024,
        ),
        name="vssm_quad_scan",
    )(x, w_in_x, w_in_z, wx_dt, wx_b, wx_c, w_dt_t, b_dt.reshape(1, E),
      a_log_t, D_param.reshape(1, E), w_out_t, gamma.reshape(1, D),
      beta.reshape(1, D))


# speedup vs baseline: 1.0368x; 1.0071x over previous
import jax
import jax.numpy as jnp
from jax.experimental import pallas as pl
from jax.experimental.pallas import tpu as pltpu

B, L, D = 8, 256, 384
E, N, R = 768, 16, 24
HW = 16
EPS = 1e-5
FILL_CHUNK = 16
UNROLL = 256
GROUPS = L // UNROLL


def _vssm_kernel(x_ref, w_in_x_ref, w_in_z_ref, wx_dt_ref, wx_b_ref,
                 wx_c_ref, w_dt_t_ref, b_dt_ref, a_log_t_ref, d_param_ref,
                 w_out_t_ref, gamma_ref, beta_ref, o_ref,
                 xin_s, z_s, dt_s, bp_s, cp_s, cpx_s, a_s,
                 abar_s, bbar_s, y0_s, y1_s, y2_s, y3_s):
    a_s[...] = -jnp.exp(a_log_t_ref[...])

    def batch_body(b, _):
        xb = x_ref[b]

        xin = jnp.dot(xb, w_in_x_ref[...], preferred_element_type=jnp.float32)
        xin_s[...] = xin
        z_s[...] = jnp.dot(xb, w_in_z_ref[...],
                           preferred_element_type=jnp.float32)
        dt_in = jnp.dot(xin, wx_dt_ref[...],
                        preferred_element_type=jnp.float32)
        bp_s[...] = jnp.dot(xin, wx_b_ref[...],
                            preferred_element_type=jnp.float32)
        cp_s[...] = jnp.dot(xin, wx_c_ref[...],
                            preferred_element_type=jnp.float32)
        cpx_s[...] = cp_s[...].reshape(L, 1, N)
        dt_s[...] = jnp.dot(dt_in, w_dt_t_ref[...],
                            preferred_element_type=jnp.float32) + b_dt_ref[...]

        def fill_body(i, _):
            sl = pl.ds(i * FILL_CHUNK, FILL_CHUNK)
            dt_r = dt_s[sl, :]
            dt_c = (jnp.maximum(dt_r, 0.0) +
                    jnp.log1p(jnp.exp(-jnp.abs(dt_r))))
            xin_c = xin_s[sl, :]
            bp_c = bp_s[sl, :]
            a_v = a_s[...]
            abar_s[sl] = jnp.exp(dt_c[:, None, :] * a_v[None, :, :])
            bbar_s[sl] = (dt_c * xin_c)[:, None, :] * bp_c[:, :, None]
            return 0

        jax.lax.fori_loop(0, L // FILL_CHUNK, fill_body, 0)

        perm = lambda t: ((t & (HW - 1)) << 4) | (t >> 4)
        loaders = (lambda i, k: UNROLL * i + k,
                   lambda i, k: (L - 1) - (UNROLL * i + k),
                   lambda i, k: perm(UNROLL * i + k),
                   lambda i, k: perm((L - 1) - (UNROLL * i + k)))
        y_refs = (y0_s, y1_s, y2_s, y3_s)

        def body(i, hs):
            hs = list(hs)
            fwd_base = pl.multiple_of(UNROLL * i, UNROLL)
            rev_base = pl.multiple_of((L - UNROLL) - UNROLL * i, UNROLL)
            for k in range(UNROLL):
                for d in range(4):
                    l = loaders[d](i, k)
                    hs[d] = abar_s[l] * hs[d] + bbar_s[l]
                    red = jnp.dot(cpx_s[l], hs[d],
                                  preferred_element_type=jnp.float32)
                    if d % 2 == 0:
                        y_refs[d][pl.ds(fwd_base + k, 1), :] = red
                    else:
                        y_refs[d][pl.ds(rev_base + (UNROLL - 1 - k), 1),
                                  :] = red
            return tuple(hs)

        z0 = jnp.zeros((N, E), jnp.float32)
        jax.lax.fori_loop(0, GROUPS, body, (z0, z0, z0, z0))

        t2 = jnp.swapaxes(y2_s[...].reshape(HW, HW, E), 0, 1).reshape(L, E)
        t3 = jnp.swapaxes(y3_s[...].reshape(HW, HW, E), 0, 1).reshape(L, E)
        ysum = y0_s[...] + y1_s[...] + t2 + t3

        z = z_s[...]
        sig = 1.0 / (1.0 + jnp.exp(-z))
        y = ysum * 0.25 * (z * sig) + xin_s[...] * d_param_ref[...]
        out = jnp.dot(y, w_out_t_ref[...],
                      preferred_element_type=jnp.float32) + xb
        mu = jnp.mean(out, axis=-1, keepdims=True)
        xc = out - mu
        var = jnp.mean(xc * xc, axis=-1, keepdims=True)
        o_ref[b] = (xc * jax.lax.rsqrt(var + EPS) * gamma_ref[...] +
                    beta_ref[...])
        return 0

    jax.lax.fori_loop(0, B, batch_body, 0)


@jax.jit
def kernel(x, W_in, A_log, W_x, W_dt, b_dt, D_param, W_out, gamma, beta):
    w_in_x = W_in[:E].T
    w_in_z = W_in[E:].T
    wx_dt = W_x[:R].T
    wx_b = W_x[R:R + N].T
    wx_c = W_x[R + N:].T
    w_dt_t = W_dt.T
    a_log_t = A_log.T
    w_out_t = W_out.T

    vmem = pl.BlockSpec(memory_space=pltpu.VMEM)
    f32 = jnp.float32
    return pl.pallas_call(
        _vssm_kernel,
        out_shape=jax.ShapeDtypeStruct((B, L, D), f32),
        in_specs=[vmem] * 13,
        out_specs=vmem,
        scratch_shapes=[
            pltpu.VMEM((L, E), f32),
            pltpu.VMEM((L, E), f32),
            pltpu.VMEM((L, E), f32),
            pltpu.VMEM((L, N), f32),
            pltpu.VMEM((L, N), f32),
            pltpu.VMEM((L, 1, N), f32),
            pltpu.VMEM((N, E), f32),
            pltpu.VMEM((L, N, E), f32),
            pltpu.VMEM((L, N, E), f32),
            pltpu.VMEM((L, E), f32),
            pltpu.VMEM((L, E), f32),
            pltpu.VMEM((L, E), f32),
            pltpu.VMEM((L, E), f32),
        ],
        compiler_params=pltpu.CompilerParams(
            vmem_limit_bytes=56 * 1024 * 1024,
        ),
        name="vssm_quad_scan",
    )(x, w_in_x, w_in_z, wx_dt, wx_b, wx_c, w_dt_t, b_dt.reshape(1, E),
      a_log_t, D_param.reshape(1, E), w_out_t, gamma.reshape(1, D),
      beta.reshape(1, D))
